# Initial kernel scaffold; baseline (speedup 1.0000x reference)
import jax
import jax.numpy as jnp
from jax.experimental import pallas as pl


def kernel(x, inv_grid):
    raise NotImplementedError("write your pallas kernel here")


# speedup vs baseline: 94.3531x; 94.3531x over previous
import functools

import jax
import jax.numpy as jnp
from jax import lax
from jax.experimental import pallas as pl
from jax.experimental.pallas import tpu as pltpu
from jax.experimental.pallas import tpu_sc as plsc

B, C, H, W = 4, 96, 224, 224
HW = H * W
NW = 32
TILES_PER_B = NW // B
CH_PER_TILE = C // TILES_PER_B
CHUNK = 6272
NCHUNK = HW // CHUNK
VPC = CHUNK // 16

def _grid_kernel(x_hbm, ig0_hbm, ig1_hbm, out_hbm, acc, xbuf, igb0, igb1):
    _F1 = jnp.float32(1.0)
    _FH = jnp.float32(0.5)
    cid = lax.axis_index("c")
    sid = lax.axis_index("s")
    wid = sid * 2 + cid
    b = wid // TILES_PER_B
    ch0 = wid % TILES_PER_B

    def pair_body(k, _):
        bch = b * C + ch0 + TILES_PER_B * k

        zeros = jnp.zeros((16,), jnp.float32)

        def zero_body(i, _):
            base = i * 64
            acc[pl.ds(base, 16)] = zeros
            acc[pl.ds(base + 16, 16)] = zeros
            acc[pl.ds(base + 32, 16)] = zeros
            acc[pl.ds(base + 48, 16)] = zeros
            return 0

        lax.fori_loop(0, HW // 64, zero_body, 0)

        pltpu.sync_copy(x_hbm.at[bch], xbuf)

        def chunk_body(cidx, _):
            start = cidx * CHUNK
            pltpu.sync_copy(ig0_hbm.at[b, pl.ds(start, CHUNK)], igb0)
            pltpu.sync_copy(ig1_hbm.at[b, pl.ds(start, CHUNK)], igb1)

            def vreg_body(v, _):
                off = v * 16
                v0 = igb0[pl.ds(off, 16)]
                v1 = igb1[pl.ds(off, 16)]
                xv = xbuf[pl.ds(start + off, 16)]

                gi = (v0 + _F1) * _FH * jnp.float32(H) + _F1
                gi = jnp.minimum(jnp.maximum(gi, jnp.float32(0.0)),
                                 jnp.float32(H + 1))
                gj = (v1 + _F1) * _FH * jnp.float32(W) + _F1
                gj = jnp.minimum(jnp.maximum(gj, jnp.float32(0.0)),
                                 jnp.float32(W + 1))
                li = gi.astype(jnp.int32)
                lj = gj.astype(jnp.int32)
                lif = li.astype(jnp.float32)
                ljf = lj.astype(jnp.float32)
                wi0 = _F1 - (gi - lif)
                wi1 = _F1 - ((lif + _F1) - gi)
                wj0 = _F1 - (gj - ljf)
                wj1 = _F1 - ((ljf + _F1) - gj)

                r0 = li - 1
                c0 = lj - 1
                idx00 = r0 * W + c0
                idx01 = idx00 + 1
                idx10 = idx00 + W
                idx11 = idx00 + (W + 1)
                rv0 = (r0 >= 0) & (r0 < H)
                rv1 = (r0 >= -1) & (r0 < H - 1)
                cv0 = (c0 >= 0) & (c0 < W)
                cv1 = (c0 >= -1) & (c0 < W - 1)
                zero_i = jnp.zeros((16,), jnp.int32)
                hi_i = jnp.int32(HW - 1)
                for idx, m, wgt in (
                    (idx00, rv0 & cv0, wi0 * wj0),
                    (idx01, rv0 & cv1, wi0 * wj1),
                    (idx10, rv1 & cv0, wi1 * wj0),
                    (idx11, rv1 & cv1, wi1 * wj1),
                ):
                    idx_c = jnp.minimum(jnp.maximum(idx, zero_i), hi_i)
                    plsc.addupdate_scatter(acc, [idx_c], xv * wgt, mask=m)
                return 0

            lax.fori_loop(0, VPC, vreg_body, 0)
            return 0

        lax.fori_loop(0, NCHUNK, chunk_body, 0)

        pltpu.sync_copy(acc, out_hbm.at[bch])
        return 0

    lax.fori_loop(0, CH_PER_TILE, pair_body, 0)


@jax.jit
def kernel(x, inv_grid):
    x2d = x.reshape(B * C, HW)
    ig0 = inv_grid[..., 0].reshape(B, HW)
    ig1 = inv_grid[..., 1].reshape(B, HW)
    mesh = plsc.VectorSubcoreMesh(core_axis_name="c", subcore_axis_name="s")
    out = pl.kernel(
        _grid_kernel,
        out_type=jax.ShapeDtypeStruct((B * C, HW), jnp.float32),
        mesh=mesh,
        compiler_params=pltpu.CompilerParams(needs_layout_passes=False),
        scratch_types=[
            pltpu.VMEM((HW,), jnp.float32),
            pltpu.VMEM((HW,), jnp.float32),
            pltpu.VMEM((CHUNK,), jnp.float32),
            pltpu.VMEM((CHUNK,), jnp.float32),
        ],
    )(x2d, ig0, ig1)
    return out.reshape(B, C, H, W)


# speedup vs baseline: 148.3451x; 1.5722x over previous
import jax
import jax.numpy as jnp
from jax import lax
from jax.experimental import pallas as pl
from jax.experimental.pallas import tpu as pltpu
from jax.experimental.pallas import tpu_sc as plsc

B, C, H, W = 4, 96, 224, 224
HW = H * W
NW = 32
TILES_PER_B = NW // B
NPASS = 6
AW = W + 2
ACC_N = 51328
CHUNK = 3136
NCHUNK = HW // CHUNK
VPC = CHUNK // 16


def _grid_kernel(x_hbm, ig0_hbm, ig1_hbm, out_hbm, acc0, acc1,
                 ig0a, ig1a, x0a, x1a, ig0b, ig1b, x0b, x1b,
                 sem0, sem1, semo):
    _F1 = jnp.float32(1.0)
    _FH = jnp.float32(0.5)
    cid = lax.axis_index("c")
    sid = lax.axis_index("s")
    wid = sid * 2 + cid
    b = wid // TILES_PER_B
    ch0 = wid % TILES_PER_B

    slots = ((ig0a, ig1a, x0a, x1a, sem0), (ig0b, ig1b, x0b, x1b, sem1))

    def issue(cidx, bchA, bchB, slot):
        g0, g1, xa, xb, sem = slot
        start = cidx * CHUNK
        pltpu.async_copy(ig0_hbm.at[pl.ds(b * HW + start, CHUNK)], g0, sem)
        pltpu.async_copy(ig1_hbm.at[pl.ds(b * HW + start, CHUNK)], g1, sem)
        pltpu.async_copy(x_hbm.at[pl.ds(bchA * HW + start, CHUNK)], xa, sem)
        pltpu.async_copy(x_hbm.at[pl.ds(bchB * HW + start, CHUNK)], xb, sem)

    def drain(slot):
        g0, g1, xa, xb, sem = slot
        for r in (g0, g1, xa, xb):
            pltpu.make_async_copy(ig0_hbm.at[pl.ds(0, CHUNK)], r, sem).wait()

    def compute(slot):
        g0, g1, xa, xb, _ = slot

        def vreg_body(v, _c):
            off = v * 16
            v0 = g0[pl.ds(off, 16)]
            v1 = g1[pl.ds(off, 16)]
            xA = xa[pl.ds(off, 16)]
            xB = xb[pl.ds(off, 16)]

            gi = (v0 + _F1) * _FH * jnp.float32(H) + _F1
            gi = jnp.minimum(jnp.maximum(gi, jnp.float32(0.0)),
                             jnp.float32(H + 1))
            gj = (v1 + _F1) * _FH * jnp.float32(W) + _F1
            gj = jnp.minimum(jnp.maximum(gj, jnp.float32(0.0)),
                             jnp.float32(W + 1))
            li = gi.astype(jnp.int32)
            lj = gj.astype(jnp.int32)
            lif = li.astype(jnp.float32)
            ljf = lj.astype(jnp.float32)
            wi0 = _F1 - (gi - lif)
            wi1 = _F1 - ((lif + _F1) - gi)
            wj0 = _F1 - (gj - ljf)
            wj1 = _F1 - ((ljf + _F1) - gj)

            idx00 = li * AW + lj
            idx01 = idx00 + 1
            idx10 = idx00 + AW
            idx11 = idx00 + (AW + 1)

            w00 = wi0 * wj0
            w01 = wi0 * wj1
            w10 = wi1 * wj0
            w11 = wi1 * wj1
            plsc.addupdate_scatter(acc0, [idx00], xA * w00)
            plsc.addupdate_scatter(acc0, [idx01], xA * w01)
            plsc.addupdate_scatter(acc0, [idx10], xA * w10)
            plsc.addupdate_scatter(acc0, [idx11], xA * w11)
            plsc.addupdate_scatter(acc1, [idx00], xB * w00)
            plsc.addupdate_scatter(acc1, [idx01], xB * w01)
            plsc.addupdate_scatter(acc1, [idx10], xB * w10)
            plsc.addupdate_scatter(acc1, [idx11], xB * w11)
            return 0

        lax.fori_loop(0, VPC, vreg_body, 0)

    def drain_out(bchA, bchB):
        pltpu.make_async_copy(acc0.at[pl.ds(0, HW)],
                              out_hbm.at[pl.ds(bchA * HW, HW)], semo).wait()
        pltpu.make_async_copy(acc1.at[pl.ds(0, HW)],
                              out_hbm.at[pl.ds(bchB * HW, HW)], semo).wait()

    def pass_body(k, _c):
        bchA = b * C + ch0 + 16 * k
        bchB = bchA + 8

        @pl.when(k > 0)
        def _():
            drain_out(bchA, bchB)

        issue(0, bchA, bchB, slots[0])

        zeros = jnp.zeros((16,), jnp.float32)

        def zero_body(i, _z):
            base = i * 64
            for q in range(4):
                acc0[pl.ds(base + 16 * q, 16)] = zeros
                acc1[pl.ds(base + 16 * q, 16)] = zeros
            return 0

        lax.fori_loop(0, ACC_N // 64, zero_body, 0)

        def chunk_body(c2, _z):
            c = c2 * 2
            drain(slots[0])
            issue(c + 1, bchA, bchB, slots[1])
            compute(slots[0])
            drain(slots[1])

            @pl.when(c2 < NCHUNK // 2 - 1)
            def _():
                issue(c + 2, bchA, bchB, slots[0])

            compute(slots[1])
            return 0

        lax.fori_loop(0, NCHUNK // 2, chunk_body, 0)

        def compact_body(r, _z):
            rbase = (r + 1) * AW + 1
            wbase = r * W
            for j in range(W // 16):
                acc0[pl.ds(wbase + 16 * j, 16)] = acc0[pl.ds(rbase + 16 * j, 16)]
                acc1[pl.ds(wbase + 16 * j, 16)] = acc1[pl.ds(rbase + 16 * j, 16)]
            return 0

        lax.fori_loop(0, H, compact_body, 0)

        pltpu.async_copy(acc0.at[pl.ds(0, HW)],
                         out_hbm.at[pl.ds(bchA * HW, HW)], semo)
        pltpu.async_copy(acc1.at[pl.ds(0, HW)],
                         out_hbm.at[pl.ds(bchB * HW, HW)], semo)
        return 0

    lax.fori_loop(0, NPASS, pass_body, 0)
    drain_out(b * C + ch0 + 16 * (NPASS - 1), b * C + ch0 + 16 * (NPASS - 1) + 8)


@jax.jit
def kernel(x, inv_grid):
    x2d = x.reshape(B * C * HW)
    ig0 = inv_grid[..., 0].reshape(B * HW)
    ig1 = inv_grid[..., 1].reshape(B * HW)
    mesh = plsc.VectorSubcoreMesh(core_axis_name="c", subcore_axis_name="s")
    out = pl.kernel(
        _grid_kernel,
        out_type=jax.ShapeDtypeStruct((B * C * HW,), jnp.float32),
        mesh=mesh,
        compiler_params=pltpu.CompilerParams(needs_layout_passes=False),
        scratch_types=[
            pltpu.VMEM((ACC_N,), jnp.float32),
            pltpu.VMEM((ACC_N,), jnp.float32),
            pltpu.VMEM((CHUNK,), jnp.float32),
            pltpu.VMEM((CHUNK,), jnp.float32),
            pltpu.VMEM((CHUNK,), jnp.float32),
            pltpu.VMEM((CHUNK,), jnp.float32),
            pltpu.VMEM((CHUNK,), jnp.float32),
            pltpu.VMEM((CHUNK,), jnp.float32),
            pltpu.VMEM((CHUNK,), jnp.float32),
            pltpu.VMEM((CHUNK,), jnp.float32),
            pltpu.SemaphoreType.DMA,
            pltpu.SemaphoreType.DMA,
            pltpu.SemaphoreType.DMA,
        ],
    )(x2d, ig0, ig1)
    return out.reshape(B, C, H, W)


# speedup vs baseline: 190.2781x; 1.2827x over previous
import jax
import jax.numpy as jnp
from jax import lax
from jax.experimental import pallas as pl
from jax.experimental.pallas import tpu as pltpu
from jax.experimental.pallas import tpu_sc as plsc

B, C, H, W = 4, 96, 224, 224
HW = H * W
NW = 32
TILES_PER_B = NW // B
PXT = HW // TILES_PER_B
NPASS = 6
AW = W + 2
ACC_N = 51328
CHUNK = 1792
NCHUNK = HW // CHUNK
VPC = CHUNK // 16


def _grid_kernel(x_hbm, ig0_hbm, ig1_hbm,
                 out_hbm, tidx_hbm, twi_hbm, twj_hbm,
                 acc0, acc1,
                 bufa, bufb, sem0, sem1, semo):
    _F1 = jnp.float32(1.0)
    _FH = jnp.float32(0.5)
    cid = lax.axis_index("c")
    sid = lax.axis_index("s")
    wid = cid * 16 + sid
    b = wid // TILES_PER_B
    ch0 = wid % TILES_PER_B

    pxbase = b * HW + ch0 * PXT

    ACHUNK = 1568

    def phasea_chunk(ca, _c):
        astart = pxbase + ca * ACHUNK
        pltpu.sync_copy(ig0_hbm.at[pl.ds(astart, ACHUNK)],
                        bufa.at[pl.ds(0, ACHUNK)])
        pltpu.sync_copy(ig1_hbm.at[pl.ds(astart, ACHUNK)],
                        bufa.at[pl.ds(ACHUNK, ACHUNK)])

        def vreg_body(v, _z):
            off = v * 16
            v0 = bufa[pl.ds(off, 16)]
            v1 = bufa[pl.ds(ACHUNK + off, 16)]
            gi = (v0 + _F1) * _FH * jnp.float32(H) + _F1
            gi = jnp.minimum(jnp.maximum(gi, jnp.float32(0.0)),
                             jnp.float32(H + 1))
            gj = (v1 + _F1) * _FH * jnp.float32(W) + _F1
            gj = jnp.minimum(jnp.maximum(gj, jnp.float32(0.0)),
                             jnp.float32(W + 1))
            li = gi.astype(jnp.int32)
            lj = gj.astype(jnp.int32)
            lif = li.astype(jnp.float32)
            ljf = lj.astype(jnp.float32)
            bufa[pl.ds(3 * ACHUNK + off, 16)] = _F1 - (gi - lif)
            bufa[pl.ds(4 * ACHUNK + off, 16)] = _F1 - (gj - ljf)
            idx_f = plsc.bitcast(li * AW + lj, jnp.float32)
            bufa[pl.ds(2 * ACHUNK + off, 16)] = idx_f
            return 0

        lax.fori_loop(0, ACHUNK // 16, vreg_body, 0)
        pltpu.sync_copy(bufa.at[pl.ds(2 * ACHUNK, ACHUNK)],
                        tidx_hbm.at[pl.ds(astart, ACHUNK)])
        pltpu.sync_copy(bufa.at[pl.ds(3 * ACHUNK, ACHUNK)],
                        twi_hbm.at[pl.ds(astart, ACHUNK)])
        pltpu.sync_copy(bufa.at[pl.ds(4 * ACHUNK, ACHUNK)],
                        twj_hbm.at[pl.ds(astart, ACHUNK)])
        return 0

    lax.fori_loop(0, PXT // ACHUNK, phasea_chunk, 0)
    plsc.subcore_barrier()

    slots = ((bufa, sem0), (bufb, sem1))

    def issue(cidx, bchA, bchB, slot):
        buf, sem = slot
        start = cidx * CHUNK
        gstart = b * HW + start
        pltpu.async_copy(tidx_hbm.at[pl.ds(gstart, CHUNK)],
                         buf.at[pl.ds(0, CHUNK)], sem)
        pltpu.async_copy(twi_hbm.at[pl.ds(gstart, CHUNK)],
                         buf.at[pl.ds(CHUNK, CHUNK)], sem)
        pltpu.async_copy(twj_hbm.at[pl.ds(gstart, CHUNK)],
                         buf.at[pl.ds(2 * CHUNK, CHUNK)], sem)
        pltpu.async_copy(x_hbm.at[pl.ds(bchA * HW + start, CHUNK)],
                         buf.at[pl.ds(3 * CHUNK, CHUNK)], sem)
        pltpu.async_copy(x_hbm.at[pl.ds(bchB * HW + start, CHUNK)],
                         buf.at[pl.ds(4 * CHUNK, CHUNK)], sem)

    def drain(slot):
        buf, sem = slot
        for q in range(5):
            pltpu.make_async_copy(tidx_hbm.at[pl.ds(0, CHUNK)],
                                  buf.at[pl.ds(q * CHUNK, CHUNK)], sem).wait()

    def compute(slot):
        buf, _ = slot

        def vreg_body(v, _z):
            off = v * 16
            idx00 = plsc.bitcast(buf[pl.ds(off, 16)], jnp.int32)
            wi0 = buf[pl.ds(CHUNK + off, 16)]
            wj0 = buf[pl.ds(2 * CHUNK + off, 16)]
            xA = buf[pl.ds(3 * CHUNK + off, 16)]
            xB = buf[pl.ds(4 * CHUNK + off, 16)]
            wi1 = _F1 - wi0
            wj1 = _F1 - wj0
            idx01 = idx00 + 1
            idx10 = idx00 + AW
            idx11 = idx00 + (AW + 1)
            w00 = wi0 * wj0
            w01 = wi0 * wj1
            w10 = wi1 * wj0
            w11 = wi1 * wj1
            plsc.addupdate_scatter(acc0, [idx00], xA * w00)
            plsc.addupdate_scatter(acc0, [idx01], xA * w01)
            plsc.addupdate_scatter(acc0, [idx10], xA * w10)
            plsc.addupdate_scatter(acc0, [idx11], xA * w11)
            plsc.addupdate_scatter(acc1, [idx00], xB * w00)
            plsc.addupdate_scatter(acc1, [idx01], xB * w01)
            plsc.addupdate_scatter(acc1, [idx10], xB * w10)
            plsc.addupdate_scatter(acc1, [idx11], xB * w11)
            return 0

        lax.fori_loop(0, VPC, vreg_body, 0)

    def drain_out(bchA, bchB):
        pltpu.make_async_copy(acc0.at[pl.ds(0, HW)],
                              out_hbm.at[pl.ds(bchA * HW, HW)], semo).wait()
        pltpu.make_async_copy(acc1.at[pl.ds(0, HW)],
                              out_hbm.at[pl.ds(bchB * HW, HW)], semo).wait()

    def pass_body(k, _c):
        bchA = b * C + ch0 + 16 * k
        bchB = bchA + 8

        @pl.when(k > 0)
        def _():
            drain_out(bchA, bchB)

        issue(0, bchA, bchB, slots[0])

        zeros = jnp.zeros((16,), jnp.float32)

        def zero_body(i, _z):
            base = i * 64
            for q in range(4):
                acc0[pl.ds(base + 16 * q, 16)] = zeros
                acc1[pl.ds(base + 16 * q, 16)] = zeros
            return 0

        lax.fori_loop(0, ACC_N // 64, zero_body, 0)

        def chunk_body(c2, _z):
            c = c2 * 2
            drain(slots[0])
            issue(c + 1, bchA, bchB, slots[1])
            compute(slots[0])
            drain(slots[1])

            @pl.when(c2 < NCHUNK // 2 - 1)
            def _():
                issue(c + 2, bchA, bchB, slots[0])

            compute(slots[1])
            return 0

        lax.fori_loop(0, NCHUNK // 2, chunk_body, 0)

        def compact_body(r, _z):
            rbase = (r + 1) * AW + 1
            wbase = r * W
            for j in range(W // 16):
                acc0[pl.ds(wbase + 16 * j, 16)] = acc0[pl.ds(rbase + 16 * j, 16)]
                acc1[pl.ds(wbase + 16 * j, 16)] = acc1[pl.ds(rbase + 16 * j, 16)]
            return 0

        lax.fori_loop(0, H, compact_body, 0)

        pltpu.async_copy(acc0.at[pl.ds(0, HW)],
                         out_hbm.at[pl.ds(bchA * HW, HW)], semo)
        pltpu.async_copy(acc1.at[pl.ds(0, HW)],
                         out_hbm.at[pl.ds(bchB * HW, HW)], semo)
        return 0

    lax.fori_loop(0, NPASS, pass_body, 0)
    drain_out(b * C + ch0 + 16 * (NPASS - 1), b * C + ch0 + 16 * (NPASS - 1) + 8)


@jax.jit
def kernel(x, inv_grid):
    x1d = x.reshape(B * C * HW)
    ig0 = inv_grid[..., 0].reshape(B * HW)
    ig1 = inv_grid[..., 1].reshape(B * HW)
    mesh = plsc.VectorSubcoreMesh(core_axis_name="c", subcore_axis_name="s")
    out, _, _, _ = pl.kernel(
        _grid_kernel,
        out_type=(
            jax.ShapeDtypeStruct((B * C * HW,), jnp.float32),
            jax.ShapeDtypeStruct((B * HW,), jnp.float32),
            jax.ShapeDtypeStruct((B * HW,), jnp.float32),
            jax.ShapeDtypeStruct((B * HW,), jnp.float32),
        ),
        mesh=mesh,
        compiler_params=pltpu.CompilerParams(needs_layout_passes=False),
        scratch_types=[
            pltpu.VMEM((ACC_N,), jnp.float32),
            pltpu.VMEM((ACC_N,), jnp.float32),
            pltpu.VMEM((5 * CHUNK,), jnp.float32),
            pltpu.VMEM((5 * CHUNK,), jnp.float32),
            pltpu.SemaphoreType.DMA,
            pltpu.SemaphoreType.DMA,
            pltpu.SemaphoreType.DMA,
        ],
    )(x1d, ig0, ig1)
    return out.reshape(B, C, H, W)


# speedup vs baseline: 190.4563x; 1.0009x over previous
import jax
import jax.numpy as jnp
from jax import lax
from jax.experimental import pallas as pl
from jax.experimental.pallas import tpu as pltpu
from jax.experimental.pallas import tpu_sc as plsc

B, C, H, W = 4, 96, 224, 224
HW = H * W
NW = 32
TILES_PER_B = NW // B
PXT = HW // TILES_PER_B
NPASS = 6
AW = W + 2
ACC_N = 51336
CHUNK = 1792
NCHUNK = HW // CHUNK
VPC = CHUNK // 16


def _grid_kernel(x_hbm, ig0_hbm, ig1_hbm,
                 out_hbm, tidx_hbm, twi_hbm, twj_hbm,
                 acc0, acc1,
                 bufa, bufb, sem0, sem1, semo):
    _F1 = jnp.float32(1.0)
    _FH = jnp.float32(0.5)
    cid = lax.axis_index("c")
    sid = lax.axis_index("s")
    wid = cid * 16 + sid
    b = wid // TILES_PER_B
    ch0 = wid % TILES_PER_B

    pxbase = b * HW + ch0 * PXT

    ACHUNK = 1568

    def phasea_chunk(ca, _c):
        astart = pxbase + ca * ACHUNK
        pltpu.sync_copy(ig0_hbm.at[pl.ds(astart, ACHUNK)],
                        bufa.at[pl.ds(0, ACHUNK)])
        pltpu.sync_copy(ig1_hbm.at[pl.ds(astart, ACHUNK)],
                        bufa.at[pl.ds(ACHUNK, ACHUNK)])

        def vreg_body(v, _z):
            off = v * 16
            v0 = bufa[pl.ds(off, 16)]
            v1 = bufa[pl.ds(ACHUNK + off, 16)]
            gi = (v0 + _F1) * _FH * jnp.float32(H) + _F1
            gi = jnp.minimum(jnp.maximum(gi, jnp.float32(0.0)),
                             jnp.float32(H + 1))
            gj = (v1 + _F1) * _FH * jnp.float32(W) + _F1
            gj = jnp.minimum(jnp.maximum(gj, jnp.float32(0.0)),
                             jnp.float32(W + 1))
            li = gi.astype(jnp.int32)
            lj = gj.astype(jnp.int32)
            lif = li.astype(jnp.float32)
            ljf = lj.astype(jnp.float32)
            bufa[pl.ds(3 * ACHUNK + off, 16)] = _F1 - (gi - lif)
            bufa[pl.ds(4 * ACHUNK + off, 16)] = _F1 - (gj - ljf)
            idx_f = plsc.bitcast(li * AW + lj, jnp.float32)
            bufa[pl.ds(2 * ACHUNK + off, 16)] = idx_f
            return 0

        lax.fori_loop(0, ACHUNK // 16, vreg_body, 0)
        pltpu.sync_copy(bufa.at[pl.ds(2 * ACHUNK, ACHUNK)],
                        tidx_hbm.at[pl.ds(astart, ACHUNK)])
        pltpu.sync_copy(bufa.at[pl.ds(3 * ACHUNK, ACHUNK)],
                        twi_hbm.at[pl.ds(astart, ACHUNK)])
        pltpu.sync_copy(bufa.at[pl.ds(4 * ACHUNK, ACHUNK)],
                        twj_hbm.at[pl.ds(astart, ACHUNK)])
        return 0

    lax.fori_loop(0, PXT // ACHUNK, phasea_chunk, 0)
    plsc.subcore_barrier()

    slots = ((bufa, sem0), (bufb, sem1))

    def issue(cidx, bchA, bchB, slot):
        buf, sem = slot
        start = cidx * CHUNK
        gstart = b * HW + start
        pltpu.async_copy(tidx_hbm.at[pl.ds(gstart, CHUNK)],
                         buf.at[pl.ds(0, CHUNK)], sem)
        pltpu.async_copy(twi_hbm.at[pl.ds(gstart, CHUNK)],
                         buf.at[pl.ds(CHUNK, CHUNK)], sem)
        pltpu.async_copy(twj_hbm.at[pl.ds(gstart, CHUNK)],
                         buf.at[pl.ds(2 * CHUNK, CHUNK)], sem)
        pltpu.async_copy(x_hbm.at[pl.ds(bchA * HW + start, CHUNK)],
                         buf.at[pl.ds(3 * CHUNK, CHUNK)], sem)
        pltpu.async_copy(x_hbm.at[pl.ds(bchB * HW + start, CHUNK)],
                         buf.at[pl.ds(4 * CHUNK, CHUNK)], sem)

    def drain(slot):
        buf, sem = slot
        for q in range(5):
            pltpu.make_async_copy(tidx_hbm.at[pl.ds(0, CHUNK)],
                                  buf.at[pl.ds(q * CHUNK, CHUNK)], sem).wait()

    def compute(slot):
        buf, _ = slot

        def vreg_body(v, _z):
            for u in range(2):
                off = v * 32 + u * 16
                idx00 = plsc.bitcast(buf[pl.ds(off, 16)], jnp.int32)
                wi0 = buf[pl.ds(CHUNK + off, 16)]
                wj0 = buf[pl.ds(2 * CHUNK + off, 16)]
                xA = buf[pl.ds(3 * CHUNK + off, 16)]
                xB = buf[pl.ds(4 * CHUNK + off, 16)]
                wi1 = _F1 - wi0
                wj1 = _F1 - wj0
                idx01 = idx00 + 1
                idx10 = idx00 + AW
                idx11 = idx00 + (AW + 1)
                w00 = wi0 * wj0
                w01 = wi0 * wj1
                w10 = wi1 * wj0
                w11 = wi1 * wj1
                plsc.addupdate_scatter(acc0, [idx00], xA * w00)
                plsc.addupdate_scatter(acc0, [idx01], xA * w01)
                plsc.addupdate_scatter(acc0, [idx10], xA * w10)
                plsc.addupdate_scatter(acc0, [idx11], xA * w11)
                plsc.addupdate_scatter(acc1, [idx00], xB * w00)
                plsc.addupdate_scatter(acc1, [idx01], xB * w01)
                plsc.addupdate_scatter(acc1, [idx10], xB * w10)
                plsc.addupdate_scatter(acc1, [idx11], xB * w11)
            return 0

        lax.fori_loop(0, VPC // 2, vreg_body, 0)

    def drain_out(bchA, bchB):
        pltpu.make_async_copy(acc0.at[pl.ds(0, HW)],
                              out_hbm.at[pl.ds(bchA * HW, HW)], semo).wait()
        pltpu.make_async_copy(acc1.at[pl.ds(0, HW)],
                              out_hbm.at[pl.ds(bchB * HW, HW)], semo).wait()

    def pass_body(k, _c):
        bchA = b * C + ch0 + 16 * k
        bchB = bchA + 8

        @pl.when(k > 0)
        def _():
            drain_out(bchA, bchB)

        issue(0, bchA, bchB, slots[0])

        zeros = jnp.zeros((16,), jnp.float32)

        def zero_body(i, _z):
            base = i * 64
            for q in range(4):
                acc0[pl.ds(base + 16 * q, 16)] = zeros
                acc1[pl.ds(base + 16 * q, 16)] = zeros
            return 0

        lax.fori_loop(0, 51328 // 64, zero_body, 0)

        def chunk_body(c2, _z):
            c = c2 * 2
            drain(slots[0])
            issue(c + 1, bchA, bchB, slots[1])
            compute(slots[0])
            drain(slots[1])

            @pl.when(c2 < NCHUNK // 2 - 1)
            def _():
                issue(c + 2, bchA, bchB, slots[0])

            compute(slots[1])
            return 0

        lax.fori_loop(0, NCHUNK // 2, chunk_body, 0)

        def compact_body(r, _z):
            rbase = (r + 1) * AW + 1
            wbase = r * W
            for j in range(W // 16):
                acc0[pl.ds(wbase + 16 * j, 16)] = acc0[pl.ds(rbase + 16 * j, 16)]
                acc1[pl.ds(wbase + 16 * j, 16)] = acc1[pl.ds(rbase + 16 * j, 16)]
            return 0

        lax.fori_loop(0, H, compact_body, 0)

        pltpu.async_copy(acc0.at[pl.ds(0, HW)],
                         out_hbm.at[pl.ds(bchA * HW, HW)], semo)
        pltpu.async_copy(acc1.at[pl.ds(0, HW)],
                         out_hbm.at[pl.ds(bchB * HW, HW)], semo)
        return 0

    lax.fori_loop(0, NPASS, pass_body, 0)
    drain_out(b * C + ch0 + 16 * (NPASS - 1), b * C + ch0 + 16 * (NPASS - 1) + 8)


@jax.jit
def kernel(x, inv_grid):
    x1d = x.reshape(B * C * HW)
    ig0 = inv_grid[..., 0].reshape(B * HW)
    ig1 = inv_grid[..., 1].reshape(B * HW)
    mesh = plsc.VectorSubcoreMesh(core_axis_name="c", subcore_axis_name="s")
    out, _, _, _ = pl.kernel(
        _grid_kernel,
        out_type=(
            jax.ShapeDtypeStruct((B * C * HW,), jnp.float32),
            jax.ShapeDtypeStruct((B * HW,), jnp.float32),
            jax.ShapeDtypeStruct((B * HW,), jnp.float32),
            jax.ShapeDtypeStruct((B * HW,), jnp.float32),
        ),
        mesh=mesh,
        compiler_params=pltpu.CompilerParams(needs_layout_passes=False),
        scratch_types=[
            pltpu.VMEM((ACC_N,), jnp.float32),
            pltpu.VMEM((ACC_N,), jnp.float32),
            pltpu.VMEM((5 * CHUNK,), jnp.float32),
            pltpu.VMEM((5 * CHUNK,), jnp.float32),
            pltpu.SemaphoreType.DMA,
            pltpu.SemaphoreType.DMA,
            pltpu.SemaphoreType.DMA,
        ],
    )(x1d, ig0, ig1)
    return out.reshape(B, C, H, W)


# speedup vs baseline: 208.1335x; 1.0928x over previous
import jax
import jax.numpy as jnp
from jax import lax
from jax.experimental import pallas as pl
from jax.experimental.pallas import tpu as pltpu
from jax.experimental.pallas import tpu_sc as plsc

B, C, H, W = 4, 96, 224, 224
HW = H * W
NW = 32
TILES_PER_B = NW // B
PXT = HW // TILES_PER_B
NPASS = 6
AW = W + 2
ACC_N = 51336
CHUNK = 1792
NCHUNK = HW // CHUNK
VPC = CHUNK // 16


def _grid_kernel(x_hbm, ig0_hbm, ig1_hbm,
                 out_hbm, tidx_hbm, twi_hbm, twj_hbm,
                 acc0, acc1,
                 bufa, bufb, sem0, sem1, semo):
    _F1 = jnp.float32(1.0)
    _FH = jnp.float32(0.5)
    cid = lax.axis_index("c")
    sid = lax.axis_index("s")
    wid = cid * 16 + sid
    b = wid // TILES_PER_B
    ch0 = wid % TILES_PER_B

    pxbase = b * HW + ch0 * PXT

    ACHUNK = 1568

    def phasea_chunk(ca, _c):
        astart = pxbase + ca * ACHUNK
        pltpu.sync_copy(ig0_hbm.at[pl.ds(astart, ACHUNK)],
                        bufa.at[pl.ds(0, ACHUNK)])
        pltpu.sync_copy(ig1_hbm.at[pl.ds(astart, ACHUNK)],
                        bufa.at[pl.ds(ACHUNK, ACHUNK)])

        def vreg_body(v, _z):
            off = v * 16
            v0 = bufa[pl.ds(off, 16)]
            v1 = bufa[pl.ds(ACHUNK + off, 16)]
            gi = (v0 + _F1) * _FH * jnp.float32(H) + _F1
            gi = jnp.minimum(jnp.maximum(gi, jnp.float32(0.0)),
                             jnp.float32(H + 1))
            gj = (v1 + _F1) * _FH * jnp.float32(W) + _F1
            gj = jnp.minimum(jnp.maximum(gj, jnp.float32(0.0)),
                             jnp.float32(W + 1))
            li = gi.astype(jnp.int32)
            lj = gj.astype(jnp.int32)
            lif = li.astype(jnp.float32)
            ljf = lj.astype(jnp.float32)
            bufa[pl.ds(3 * ACHUNK + off, 16)] = _F1 - (gi - lif)
            bufa[pl.ds(4 * ACHUNK + off, 16)] = _F1 - (gj - ljf)
            idx_f = plsc.bitcast(li * AW + lj, jnp.float32)
            bufa[pl.ds(2 * ACHUNK + off, 16)] = idx_f
            return 0

        lax.fori_loop(0, ACHUNK // 16, vreg_body, 0)
        pltpu.sync_copy(bufa.at[pl.ds(2 * ACHUNK, ACHUNK)],
                        tidx_hbm.at[pl.ds(astart, ACHUNK)])
        pltpu.sync_copy(bufa.at[pl.ds(3 * ACHUNK, ACHUNK)],
                        twi_hbm.at[pl.ds(astart, ACHUNK)])
        pltpu.sync_copy(bufa.at[pl.ds(4 * ACHUNK, ACHUNK)],
                        twj_hbm.at[pl.ds(astart, ACHUNK)])
        return 0

    lax.fori_loop(0, PXT // ACHUNK, phasea_chunk, 0)
    plsc.subcore_barrier()

    slots = ((bufa, sem0), (bufb, sem1))

    def issue(cidx, bchA, bchB, slot):
        buf, sem = slot
        start = cidx * CHUNK
        gstart = b * HW + start
        pltpu.async_copy(tidx_hbm.at[pl.ds(gstart, CHUNK)],
                         buf.at[pl.ds(0, CHUNK)], sem)
        pltpu.async_copy(twi_hbm.at[pl.ds(gstart, CHUNK)],
                         buf.at[pl.ds(CHUNK, CHUNK)], sem)
        pltpu.async_copy(twj_hbm.at[pl.ds(gstart, CHUNK)],
                         buf.at[pl.ds(2 * CHUNK, CHUNK)], sem)
        pltpu.async_copy(x_hbm.at[pl.ds(bchA * HW + start, CHUNK)],
                         buf.at[pl.ds(3 * CHUNK, CHUNK)], sem)
        pltpu.async_copy(x_hbm.at[pl.ds(bchB * HW + start, CHUNK)],
                         buf.at[pl.ds(4 * CHUNK, CHUNK)], sem)

    def drain(slot):
        buf, sem = slot
        for q in range(5):
            pltpu.make_async_copy(tidx_hbm.at[pl.ds(0, CHUNK)],
                                  buf.at[pl.ds(q * CHUNK, CHUNK)], sem).wait()

    def compute(slot):
        buf, _ = slot

        @plsc.parallel_loop(0, VPC // 2, unroll=1)
        def vreg_body(v):
            for u in range(2):
                off = v * 32 + u * 16
                idx00 = plsc.bitcast(buf[pl.ds(off, 16)], jnp.int32)
                wi0 = buf[pl.ds(CHUNK + off, 16)]
                wj0 = buf[pl.ds(2 * CHUNK + off, 16)]
                xA = buf[pl.ds(3 * CHUNK + off, 16)]
                xB = buf[pl.ds(4 * CHUNK + off, 16)]
                wi1 = _F1 - wi0
                wj1 = _F1 - wj0
                idx01 = idx00 + 1
                idx10 = idx00 + AW
                idx11 = idx00 + (AW + 1)
                w00 = wi0 * wj0
                w01 = wi0 * wj1
                w10 = wi1 * wj0
                w11 = wi1 * wj1
                plsc.addupdate_scatter(acc0, [idx00], xA * w00)
                plsc.addupdate_scatter(acc0, [idx01], xA * w01)
                plsc.addupdate_scatter(acc0, [idx10], xA * w10)
                plsc.addupdate_scatter(acc0, [idx11], xA * w11)
                plsc.addupdate_scatter(acc1, [idx00], xB * w00)
                plsc.addupdate_scatter(acc1, [idx01], xB * w01)
                plsc.addupdate_scatter(acc1, [idx10], xB * w10)
                plsc.addupdate_scatter(acc1, [idx11], xB * w11)

    def drain_out(bchA, bchB):
        pltpu.make_async_copy(acc0.at[pl.ds(0, HW)],
                              out_hbm.at[pl.ds(bchA * HW, HW)], semo).wait()
        pltpu.make_async_copy(acc1.at[pl.ds(0, HW)],
                              out_hbm.at[pl.ds(bchB * HW, HW)], semo).wait()

    def pass_body(k, _c):
        bchA = b * C + ch0 + 16 * k
        bchB = bchA + 8

        @pl.when(k > 0)
        def _():
            drain_out(bchA, bchB)

        issue(0, bchA, bchB, slots[0])

        zeros = jnp.zeros((16,), jnp.float32)

        def zero_body(i, _z):
            base = i * 64
            for q in range(4):
                acc0[pl.ds(base + 16 * q, 16)] = zeros
                acc1[pl.ds(base + 16 * q, 16)] = zeros
            return 0

        lax.fori_loop(0, 51328 // 64, zero_body, 0)

        def chunk_body(c2, _z):
            c = c2 * 2
            drain(slots[0])
            issue(c + 1, bchA, bchB, slots[1])
            compute(slots[0])
            drain(slots[1])

            @pl.when(c2 < NCHUNK // 2 - 1)
            def _():
                issue(c + 2, bchA, bchB, slots[0])

            compute(slots[1])
            return 0

        lax.fori_loop(0, NCHUNK // 2, chunk_body, 0)

        def compact_body(r, _z):
            rbase = (r + 1) * AW + 1
            wbase = r * W
            for j in range(W // 16):
                acc0[pl.ds(wbase + 16 * j, 16)] = acc0[pl.ds(rbase + 16 * j, 16)]
                acc1[pl.ds(wbase + 16 * j, 16)] = acc1[pl.ds(rbase + 16 * j, 16)]
            return 0

        lax.fori_loop(0, H, compact_body, 0)

        pltpu.async_copy(acc0.at[pl.ds(0, HW)],
                         out_hbm.at[pl.ds(bchA * HW, HW)], semo)
        pltpu.async_copy(acc1.at[pl.ds(0, HW)],
                         out_hbm.at[pl.ds(bchB * HW, HW)], semo)
        return 0

    lax.fori_loop(0, NPASS, pass_body, 0)
    drain_out(b * C + ch0 + 16 * (NPASS - 1), b * C + ch0 + 16 * (NPASS - 1) + 8)


@jax.jit
def kernel(x, inv_grid):
    x1d = x.reshape(B * C * HW)
    ig0 = inv_grid[..., 0].reshape(B * HW)
    ig1 = inv_grid[..., 1].reshape(B * HW)
    mesh = plsc.VectorSubcoreMesh(core_axis_name="c", subcore_axis_name="s")
    out, _, _, _ = pl.kernel(
        _grid_kernel,
        out_type=(
            jax.ShapeDtypeStruct((B * C * HW,), jnp.float32),
            jax.ShapeDtypeStruct((B * HW,), jnp.float32),
            jax.ShapeDtypeStruct((B * HW,), jnp.float32),
            jax.ShapeDtypeStruct((B * HW,), jnp.float32),
        ),
        mesh=mesh,
        compiler_params=pltpu.CompilerParams(needs_layout_passes=False),
        scratch_types=[
            pltpu.VMEM((ACC_N,), jnp.float32),
            pltpu.VMEM((ACC_N,), jnp.float32),
            pltpu.VMEM((5 * CHUNK,), jnp.float32),
            pltpu.VMEM((5 * CHUNK,), jnp.float32),
            pltpu.SemaphoreType.DMA,
            pltpu.SemaphoreType.DMA,
            pltpu.SemaphoreType.DMA,
        ],
    )(x1d, ig0, ig1)
    return out.reshape(B, C, H, W)
